# baseline (speedup 1.0000x reference)
import functools

import jax
import jax.numpy as jnp
from jax import lax
from jax.experimental import pallas as pl
from jax.experimental.pallas import tpu as pltpu
from jax.experimental.pallas import tpu_sc as plsc

N = 10000
E = 320000
D = 128
NC = 2
NS = 16
NW = NC * NS

CHUNK = 128
PCHUNKS = 2560
CPW = PCHUNKS // NW
PAIRS = CPW // 2
E_PAD = PCHUNKS * CHUNK

N_PAD = 10112
ROWS_TILE = N_PAD // NS
OUT_LAST = N - ROWS_TILE * (NS - 1)
N_DEG = 10240
DEG_SLICE = N_DEG // NS


def _make_sc_agg(with_deg):
    if with_deg:
        out_type = (jax.ShapeDtypeStruct((NC, N, D), jnp.float32),
                    jax.ShapeDtypeStruct((NC, 1, N_DEG), jnp.float32))
    else:
        out_type = jax.ShapeDtypeStruct((NC, N, D), jnp.float32)

    scratch = [
        pltpu.VMEM((CHUNK, D), jnp.float32),
        pltpu.VMEM((CHUNK, D), jnp.float32),
        pltpu.VMEM((CHUNK,), jnp.int32),
        pltpu.VMEM((CHUNK,), jnp.int32),
        pltpu.VMEM((CHUNK,), jnp.int32),
        pltpu.VMEM((CHUNK,), jnp.int32),
        pltpu.VMEM_SHARED((N_PAD, D), jnp.float32),
        pltpu.SemaphoreType.DMA,
        pltpu.SemaphoreType.DMA,
        pltpu.SemaphoreType.DMA,
    ]
    if with_deg:
        scratch += [
            pltpu.VMEM((CHUNK,), jnp.float32),
            pltpu.VMEM((DEG_SLICE,), jnp.float32),
            pltpu.VMEM_SHARED((N_DEG,), jnp.float32),
        ]

    mesh = plsc.VectorSubcoreMesh(core_axis_name="c", subcore_axis_name="s")

    @functools.partial(pl.kernel, out_type=out_type, mesh=mesh,
                       scratch_types=scratch)
    def sc_agg(*refs):
        if with_deg:
            (h_hbm, src_hbm, dst_hbm, agg_hbm, deg_hbm,
             rb0, rb1, s0, d0, s1, d1, agg_sh, sem, semi0, semi1,
             onesbuf, zdbuf, deg_sh) = refs
        else:
            (h_hbm, src_hbm, dst_hbm, agg_hbm,
             rb0, rb1, s0, d0, s1, d1, agg_sh, sem, semi0, semi1) = refs

        c = lax.axis_index("c")
        s = lax.axis_index("s")
        wid = s * NC + c
        base = wid * CPW

        def idx_load(j, sbuf, dbuf, semx):
            off = (base + j) * CHUNK
            pltpu.async_copy(src_hbm.at[pl.ds(off, CHUNK)], sbuf, semx)
            pltpu.async_copy(dst_hbm.at[pl.ds(off, CHUNK)], dbuf, semx)

        def idx_wait(j, sbuf, dbuf, semx):
            off = (base + j) * CHUNK
            pltpu.make_async_copy(src_hbm.at[pl.ds(off, CHUNK)], sbuf, semx).wait()
            pltpu.make_async_copy(dst_hbm.at[pl.ds(off, CHUNK)], dbuf, semx).wait()

        idx_load(0, s0, d0, semi0)
        idx_load(1, s1, d1, semi1)

        zv = jnp.zeros((16,), jnp.float32)

        def zrow(r, carry):
            for j in range(D // 16):
                rb0[r, pl.ds(j * 16, 16)] = zv
            return carry

        lax.fori_loop(0, CHUNK, zrow, 0)

        if with_deg:
            ov = jnp.ones((16,), jnp.float32)
            for j in range(CHUNK // 16):
                onesbuf[pl.ds(j * 16, 16)] = ov
            for j in range(DEG_SLICE // 16):
                zdbuf[pl.ds(j * 16, 16)] = zv

        full, rem = ROWS_TILE // CHUNK, ROWS_TILE % CHUNK
        for k in range(full):
            pltpu.sync_copy(rb0, agg_sh.at[pl.ds(s * ROWS_TILE + k * CHUNK, CHUNK), :])
        if rem:
            pltpu.sync_copy(rb0.at[pl.ds(0, rem), :],
                            agg_sh.at[pl.ds(s * ROWS_TILE + full * CHUNK, rem), :])
        if with_deg:
            pltpu.sync_copy(zdbuf, deg_sh.at[pl.ds(s * DEG_SLICE, DEG_SLICE)])

        idx_wait(0, s0, d0, semi0)
        pltpu.async_copy(h_hbm.at[s0], rb0, sem)

        plsc.subcore_barrier()

        def pair(p, carry):
            a = 2 * p
            pltpu.make_async_copy(h_hbm.at[s0], rb0, sem).wait()
            idx_wait(a + 1, s1, d1, semi1)
            pltpu.async_copy(h_hbm.at[s1], rb1, sem)
            pltpu.sync_copy(rb0, agg_sh.at[d0], add=True)
            if with_deg:
                pltpu.sync_copy(onesbuf, deg_sh.at[d0], add=True)

            @pl.when(a + 2 < CPW)
            def _():
                idx_load(a + 2, s0, d0, semi0)

            pltpu.make_async_copy(h_hbm.at[s1], rb1, sem).wait()
            pltpu.sync_copy(rb1, agg_sh.at[d1], add=True)
            if with_deg:
                pltpu.sync_copy(onesbuf, deg_sh.at[d1], add=True)

            @pl.when(a + 3 < CPW)
            def _():
                idx_load(a + 3, s1, d1, semi1)

            @pl.when(a + 2 < CPW)
            def _():
                idx_wait(a + 2, s0, d0, semi0)
                pltpu.async_copy(h_hbm.at[s0], rb0, sem)

            return carry

        lax.fori_loop(0, PAIRS, pair, 0)

        plsc.subcore_barrier()

        @pl.when(s < NS - 1)
        def _():
            pltpu.sync_copy(agg_sh.at[pl.ds(s * ROWS_TILE, ROWS_TILE), :],
                            agg_hbm.at[c, pl.ds(s * ROWS_TILE, ROWS_TILE), :])

        @pl.when(s == NS - 1)
        def _():
            pltpu.sync_copy(agg_sh.at[pl.ds((NS - 1) * ROWS_TILE, OUT_LAST), :],
                            agg_hbm.at[c, pl.ds((NS - 1) * ROWS_TILE, OUT_LAST), :])

        if with_deg:
            pltpu.sync_copy(deg_sh.at[pl.ds(s * DEG_SLICE, DEG_SLICE)],
                            deg_hbm.at[c, 0, pl.ds(s * DEG_SLICE, DEG_SLICE)])

    return sc_agg


@functools.lru_cache(maxsize=None)
def _get_sc_agg(with_deg):
    return _make_sc_agg(with_deg)


RB = 1000
GRID = N // RB


def _combine_body(act, h_ref, agg_ref, deg_ref, ws_ref, wn_ref, b_ref, o_ref):
    deg = jnp.sum(deg_ref[...], axis=(0, 1))
    inv = (1.0 / jnp.maximum(deg, 1.0))[:, None]
    agg = agg_ref[0] + agg_ref[1]
    out = jnp.dot(h_ref[...], ws_ref[...], preferred_element_type=jnp.float32)
    out = out + jnp.dot(agg * inv, wn_ref[...], preferred_element_type=jnp.float32)
    out = out + b_ref[...]
    if act:
        out = jnp.maximum(out, 0.0)
    o_ref[...] = out


def _combine(h, agg2, deg_r, Ws, Wn, b, act):
    return pl.pallas_call(
        functools.partial(_combine_body, act),
        out_shape=jax.ShapeDtypeStruct((N, D), jnp.float32),
        grid=(GRID,),
        in_specs=[
            pl.BlockSpec((RB, D), lambda i: (i, 0)),
            pl.BlockSpec((NC, RB, D), lambda i: (0, i, 0)),
            pl.BlockSpec((1, NC, RB), lambda i: (i, 0, 0)),
            pl.BlockSpec((D, D), lambda i: (0, 0)),
            pl.BlockSpec((D, D), lambda i: (0, 0)),
            pl.BlockSpec((1, D), lambda i: (0, 0)),
        ],
        out_specs=pl.BlockSpec((RB, D), lambda i: (i, 0)),
    )(h, agg2, deg_r, Ws, Wn, b)


def kernel(features, edge_index, Wself0, Wneigh0, b0, Wself1, Wneigh1, b1,
           Wself2, Wneigh2, b2):
    src = edge_index[0]
    dst = edge_index[1]
    pad = E_PAD - E
    src_p = jnp.concatenate([src, jnp.zeros((pad,), src.dtype)])
    dst_p = jnp.concatenate([dst, jnp.full((pad,), N, dst.dtype)])

    agg0, deg2 = _get_sc_agg(True)(features, src_p, dst_p)
    deg_r = deg2[:, 0, :N].reshape(NC, GRID, RB).transpose(1, 0, 2)

    h1 = _combine(features, agg0, deg_r, Wself0, Wneigh0, b0.reshape(1, D), True)
    agg1 = _get_sc_agg(False)(h1, src_p, dst_p)
    h2 = _combine(h1, agg1, deg_r, Wself1, Wneigh1, b1.reshape(1, D), True)
    agg2 = _get_sc_agg(False)(h2, src_p, dst_p)
    h3 = _combine(h2, agg2, deg_r, Wself2, Wneigh2, b2.reshape(1, D), False)
    return h3


# speedup vs baseline: 3.0922x; 3.0922x over previous
import functools

import jax
import jax.numpy as jnp
from jax import lax
from jax.experimental import pallas as pl
from jax.experimental.pallas import tpu as pltpu
from jax.experimental.pallas import tpu_sc as plsc

N = 10000
E = 320000
D = 128
NC = 2
NS = 16
NW = NC * NS

CHUNK = 128
PCHUNKS = 2560
CPW = PCHUNKS // NW
PAIRS = CPW // 2
E_PAD = PCHUNKS * CHUNK

N_PAD = 10112
ROWS_TILE = N_PAD // NS
OUT_LAST = N - ROWS_TILE * (NS - 1)
N_DEG = 10240
DEG_SLICE = N_DEG // NS


def _make_sc_agg(with_deg):
    if with_deg:
        out_type = (jax.ShapeDtypeStruct((NC, N, D), jnp.float32),
                    jax.ShapeDtypeStruct((NC, 1, N_DEG), jnp.float32))
    else:
        out_type = jax.ShapeDtypeStruct((NC, N, D), jnp.float32)

    scratch = [
        pltpu.VMEM((CHUNK, D), jnp.float32),
        pltpu.VMEM((CHUNK, D), jnp.float32),
        pltpu.VMEM((CHUNK,), jnp.int32),
        pltpu.VMEM((CHUNK,), jnp.int32),
        pltpu.VMEM((CHUNK,), jnp.int32),
        pltpu.VMEM((CHUNK,), jnp.int32),
        pltpu.VMEM_SHARED((N_PAD, D), jnp.float32),
        pltpu.SemaphoreType.DMA,
        pltpu.SemaphoreType.DMA,
        pltpu.SemaphoreType.DMA,
    ]
    if with_deg:
        scratch += [
            pltpu.VMEM((CHUNK,), jnp.float32),
            pltpu.VMEM((DEG_SLICE,), jnp.float32),
            pltpu.VMEM_SHARED((N_DEG,), jnp.float32),
        ]

    mesh = plsc.VectorSubcoreMesh(core_axis_name="c", subcore_axis_name="s")

    @functools.partial(pl.kernel, out_type=out_type, mesh=mesh,
                       scratch_types=scratch)
    def sc_agg(*refs):
        if with_deg:
            (h_hbm, src_hbm, dst_hbm, agg_hbm, deg_hbm,
             rb0, rb1, s0, d0, s1, d1, agg_sh, sem, semi0, semi1,
             onesbuf, zdbuf, deg_sh) = refs
        else:
            (h_hbm, src_hbm, dst_hbm, agg_hbm,
             rb0, rb1, s0, d0, s1, d1, agg_sh, sem, semi0, semi1) = refs

        c = lax.axis_index("c")
        s = lax.axis_index("s")
        wid = s * NC + c
        base = wid * CPW

        def idx_load(j, sbuf, dbuf, semx):
            off = (base + j) * CHUNK
            pltpu.async_copy(src_hbm.at[pl.ds(off, CHUNK)], sbuf, semx)
            pltpu.async_copy(dst_hbm.at[pl.ds(off, CHUNK)], dbuf, semx)

        def idx_wait(j, sbuf, dbuf, semx):
            off = (base + j) * CHUNK
            pltpu.make_async_copy(src_hbm.at[pl.ds(off, CHUNK)], sbuf, semx).wait()
            pltpu.make_async_copy(dst_hbm.at[pl.ds(off, CHUNK)], dbuf, semx).wait()

        idx_load(0, s0, d0, semi0)
        idx_load(1, s1, d1, semi1)

        zv = jnp.zeros((16,), jnp.float32)

        def zrow(r, carry):
            for j in range(D // 16):
                rb0[r, pl.ds(j * 16, 16)] = zv
            return carry

        lax.fori_loop(0, CHUNK, zrow, 0)

        if with_deg:
            ov = jnp.ones((16,), jnp.float32)
            for j in range(CHUNK // 16):
                onesbuf[pl.ds(j * 16, 16)] = ov
            for j in range(DEG_SLICE // 16):
                zdbuf[pl.ds(j * 16, 16)] = zv

        full, rem = ROWS_TILE // CHUNK, ROWS_TILE % CHUNK
        for k in range(full):
            pltpu.sync_copy(rb0, agg_sh.at[pl.ds(s * ROWS_TILE + k * CHUNK, CHUNK), :])
        if rem:
            pltpu.sync_copy(rb0.at[pl.ds(0, rem), :],
                            agg_sh.at[pl.ds(s * ROWS_TILE + full * CHUNK, rem), :])
        if with_deg:
            pltpu.sync_copy(zdbuf, deg_sh.at[pl.ds(s * DEG_SLICE, DEG_SLICE)])

        idx_wait(0, s0, d0, semi0)
        pltpu.async_copy(h_hbm.at[s0], rb0, sem)

        plsc.subcore_barrier()

        def pair(p, carry):
            a = 2 * p
            pltpu.make_async_copy(h_hbm.at[s0], rb0, sem).wait()
            idx_wait(a + 1, s1, d1, semi1)
            pltpu.async_copy(h_hbm.at[s1], rb1, sem)
            pltpu.sync_copy(rb0, agg_sh.at[d0], add=True)
            if with_deg:
                pltpu.sync_copy(onesbuf, deg_sh.at[d0], add=True)

            @pl.when(a + 2 < CPW)
            def _():
                idx_load(a + 2, s0, d0, semi0)

            pltpu.make_async_copy(h_hbm.at[s1], rb1, sem).wait()
            pltpu.sync_copy(rb1, agg_sh.at[d1], add=True)
            if with_deg:
                pltpu.sync_copy(onesbuf, deg_sh.at[d1], add=True)

            @pl.when(a + 3 < CPW)
            def _():
                idx_load(a + 3, s1, d1, semi1)

            @pl.when(a + 2 < CPW)
            def _():
                idx_wait(a + 2, s0, d0, semi0)
                pltpu.async_copy(h_hbm.at[s0], rb0, sem)

            return carry

        lax.fori_loop(0, PAIRS, pair, 0)

        plsc.subcore_barrier()

        @pl.when(s < NS - 1)
        def _():
            pltpu.sync_copy(agg_sh.at[pl.ds(s * ROWS_TILE, ROWS_TILE), :],
                            agg_hbm.at[c, pl.ds(s * ROWS_TILE, ROWS_TILE), :])

        @pl.when(s == NS - 1)
        def _():
            pltpu.sync_copy(agg_sh.at[pl.ds((NS - 1) * ROWS_TILE, OUT_LAST), :],
                            agg_hbm.at[c, pl.ds((NS - 1) * ROWS_TILE, OUT_LAST), :])

        if with_deg:
            pltpu.sync_copy(deg_sh.at[pl.ds(s * DEG_SLICE, DEG_SLICE)],
                            deg_hbm.at[c, 0, pl.ds(s * DEG_SLICE, DEG_SLICE)])

    return sc_agg


@functools.lru_cache(maxsize=None)
def _get_sc_agg(with_deg):
    return _make_sc_agg(with_deg)


RB = 1000
GRID = N // RB


def _combine_body(act, h_ref, agg_ref, deg_ref, ws_ref, wn_ref, b_ref, o_ref):
    deg = jnp.sum(deg_ref[...], axis=(0, 1))
    inv = (1.0 / jnp.maximum(deg, 1.0))[:, None]
    agg = agg_ref[0] + agg_ref[1]
    out = jnp.dot(h_ref[...], ws_ref[...], preferred_element_type=jnp.float32)
    out = out + jnp.dot(agg * inv, wn_ref[...], preferred_element_type=jnp.float32)
    out = out + b_ref[...]
    if act:
        out = jnp.maximum(out, 0.0)
    o_ref[...] = out


def _combine(h, agg2, deg_r, Ws, Wn, b, act):
    return pl.pallas_call(
        functools.partial(_combine_body, act),
        out_shape=jax.ShapeDtypeStruct((N, D), jnp.float32),
        grid=(GRID,),
        in_specs=[
            pl.BlockSpec((RB, D), lambda i: (i, 0)),
            pl.BlockSpec((NC, RB, D), lambda i: (0, i, 0)),
            pl.BlockSpec((1, NC, RB), lambda i: (i, 0, 0)),
            pl.BlockSpec((D, D), lambda i: (0, 0)),
            pl.BlockSpec((D, D), lambda i: (0, 0)),
            pl.BlockSpec((1, D), lambda i: (0, 0)),
        ],
        out_specs=pl.BlockSpec((RB, D), lambda i: (i, 0)),
    )(h, agg2, deg_r, Ws, Wn, b)


def kernel(features, edge_index, Wself0, Wneigh0, b0, Wself1, Wneigh1, b1,
           Wself2, Wneigh2, b2):
    src = edge_index[0]
    dst = edge_index[1]
    pad = E_PAD - E
    ar = jnp.arange(pad, dtype=src.dtype)
    src_p = jnp.concatenate([src, ar % N])
    dst_p = jnp.concatenate([dst, N + ar % (N_PAD - N)])

    agg0, deg2 = _get_sc_agg(True)(features, src_p, dst_p)
    deg_r = deg2[:, 0, :N].reshape(NC, GRID, RB).transpose(1, 0, 2)

    h1 = _combine(features, agg0, deg_r, Wself0, Wneigh0, b0.reshape(1, D), True)
    agg1 = _get_sc_agg(False)(h1, src_p, dst_p)
    h2 = _combine(h1, agg1, deg_r, Wself1, Wneigh1, b1.reshape(1, D), True)
    agg2 = _get_sc_agg(False)(h2, src_p, dst_p)
    h3 = _combine(h2, agg2, deg_r, Wself2, Wneigh2, b2.reshape(1, D), False)
    return h3


# speedup vs baseline: 3.5790x; 1.1574x over previous
import functools

import jax
import jax.numpy as jnp
from jax import lax
from jax.experimental import pallas as pl
from jax.experimental.pallas import tpu as pltpu
from jax.experimental.pallas import tpu_sc as plsc

N = 10000
E = 320000
D = 128
NC = 2
NS = 16
NW = NC * NS

CHUNK = 128
PCHUNKS = 2560
CPW = PCHUNKS // NW
PAIRS = CPW // 2
E_PAD = PCHUNKS * CHUNK

N_PAD = 10112
ROWS_TILE = N_PAD // NS
OUT_LAST = N - ROWS_TILE * (NS - 1)
N_DEG = 10240
DEG_SLICE = N_DEG // NS


def _make_sc_agg(with_deg):
    if with_deg:
        out_type = (jax.ShapeDtypeStruct((NC, N, D), jnp.float32),
                    jax.ShapeDtypeStruct((NC, 1, N_DEG), jnp.float32))
    else:
        out_type = jax.ShapeDtypeStruct((NC, N, D), jnp.float32)

    scratch = [
        pltpu.VMEM((CHUNK, D), jnp.float32),
        pltpu.VMEM((CHUNK, D), jnp.float32),
        pltpu.VMEM((CHUNK,), jnp.int32),
        pltpu.VMEM((CHUNK,), jnp.int32),
        pltpu.VMEM((CHUNK,), jnp.int32),
        pltpu.VMEM((CHUNK,), jnp.int32),
        pltpu.VMEM((CHUNK,), jnp.int32),
        pltpu.VMEM((CHUNK,), jnp.int32),
        pltpu.VMEM((CHUNK,), jnp.int32),
        pltpu.VMEM((CHUNK,), jnp.int32),
        pltpu.VMEM_SHARED((N_PAD, D), jnp.float32),
        pltpu.SemaphoreType.DMA,
        pltpu.SemaphoreType.DMA,
        pltpu.SemaphoreType.DMA,
        pltpu.SemaphoreType.DMA,
        pltpu.SemaphoreType.DMA,
    ]
    if with_deg:
        scratch += [
            pltpu.VMEM((CHUNK,), jnp.float32),
            pltpu.VMEM((DEG_SLICE,), jnp.float32),
            pltpu.VMEM_SHARED((N_DEG,), jnp.float32),
        ]

    mesh = plsc.VectorSubcoreMesh(core_axis_name="c", subcore_axis_name="s")

    @functools.partial(pl.kernel, out_type=out_type, mesh=mesh,
                       scratch_types=scratch)
    def sc_agg(*refs):
        if with_deg:
            (h_hbm, src_hbm, dst_hbm, agg_hbm, deg_hbm,
             rb0, rb1, s0, d0, s1, d1, s2, d2, s3, d3, agg_sh,
             sem, semi0, semi1, semi2, semi3,
             onesbuf, zdbuf, deg_sh) = refs
        else:
            (h_hbm, src_hbm, dst_hbm, agg_hbm,
             rb0, rb1, s0, d0, s1, d1, s2, d2, s3, d3, agg_sh,
             sem, semi0, semi1, semi2, semi3) = refs
        isets = [(s0, d0, semi0), (s1, d1, semi1),
                 (s2, d2, semi2), (s3, d3, semi3)]

        c = lax.axis_index("c")
        s = lax.axis_index("s")
        wid = s * NC + c
        base = wid * CPW

        def idx_load(j, sbuf, dbuf, semx):
            off = (base + j) * CHUNK
            pltpu.async_copy(src_hbm.at[pl.ds(off, CHUNK)], sbuf, semx)
            pltpu.async_copy(dst_hbm.at[pl.ds(off, CHUNK)], dbuf, semx)

        def idx_wait(j, sbuf, dbuf, semx):
            off = (base + j) * CHUNK
            pltpu.make_async_copy(src_hbm.at[pl.ds(off, CHUNK)], sbuf, semx).wait()
            pltpu.make_async_copy(dst_hbm.at[pl.ds(off, CHUNK)], dbuf, semx).wait()

        for j in range(4):
            idx_load(j, *isets[j])

        zv = jnp.zeros((16,), jnp.float32)

        def zrow(r, carry):
            for j in range(D // 16):
                rb0[r, pl.ds(j * 16, 16)] = zv
            return carry

        lax.fori_loop(0, CHUNK, zrow, 0)

        if with_deg:
            ov = jnp.ones((16,), jnp.float32)
            for j in range(CHUNK // 16):
                onesbuf[pl.ds(j * 16, 16)] = ov
            for j in range(DEG_SLICE // 16):
                zdbuf[pl.ds(j * 16, 16)] = zv

        full, rem = ROWS_TILE // CHUNK, ROWS_TILE % CHUNK
        for k in range(full):
            pltpu.sync_copy(rb0, agg_sh.at[pl.ds(s * ROWS_TILE + k * CHUNK, CHUNK), :])
        if rem:
            pltpu.sync_copy(rb0.at[pl.ds(0, rem), :],
                            agg_sh.at[pl.ds(s * ROWS_TILE + full * CHUNK, rem), :])
        if with_deg:
            pltpu.sync_copy(zdbuf, deg_sh.at[pl.ds(s * DEG_SLICE, DEG_SLICE)])

        idx_wait(0, *isets[0])
        pltpu.async_copy(h_hbm.at[s0], rb0, sem)

        plsc.subcore_barrier()

        rbs = [rb0, rb1]

        def quad(q, carry):
            c0 = 4 * q
            for k in range(4):
                ck = c0 + k
                sk, dk, semk = isets[k]
                rbk = rbs[k % 2]
                pltpu.make_async_copy(h_hbm.at[sk], rbk, sem).wait()
                if k < 3:
                    sn, dn, semn = isets[k + 1]
                    idx_wait(ck + 1, sn, dn, semn)
                    pltpu.async_copy(h_hbm.at[sn], rbs[(k + 1) % 2], sem)
                else:
                    @pl.when(ck + 1 < CPW)
                    def _():
                        sn, dn, semn = isets[0]
                        idx_wait(ck + 1, sn, dn, semn)
                        pltpu.async_copy(h_hbm.at[sn], rbs[0], sem)
                pltpu.sync_copy(rbk, agg_sh.at[dk], add=True)
                if with_deg:
                    pltpu.sync_copy(onesbuf, deg_sh.at[dk], add=True)
                @pl.when(ck + 4 < CPW)
                def _():
                    idx_load(ck + 4, sk, dk, semk)
            return carry

        lax.fori_loop(0, CPW // 4, quad, 0)

        plsc.subcore_barrier()

        @pl.when(s < NS - 1)
        def _():
            pltpu.sync_copy(agg_sh.at[pl.ds(s * ROWS_TILE, ROWS_TILE), :],
                            agg_hbm.at[c, pl.ds(s * ROWS_TILE, ROWS_TILE), :])

        @pl.when(s == NS - 1)
        def _():
            pltpu.sync_copy(agg_sh.at[pl.ds((NS - 1) * ROWS_TILE, OUT_LAST), :],
                            agg_hbm.at[c, pl.ds((NS - 1) * ROWS_TILE, OUT_LAST), :])

        if with_deg:
            pltpu.sync_copy(deg_sh.at[pl.ds(s * DEG_SLICE, DEG_SLICE)],
                            deg_hbm.at[c, 0, pl.ds(s * DEG_SLICE, DEG_SLICE)])

    return sc_agg


@functools.lru_cache(maxsize=None)
def _get_sc_agg(with_deg):
    return _make_sc_agg(with_deg)


RB = 1000
GRID = N // RB


def _combine_body(act, h_ref, agg_ref, deg_ref, ws_ref, wn_ref, b_ref, o_ref):
    deg = jnp.sum(deg_ref[...], axis=(0, 1))
    inv = (1.0 / jnp.maximum(deg, 1.0))[:, None]
    agg = agg_ref[0] + agg_ref[1]
    out = jnp.dot(h_ref[...], ws_ref[...], preferred_element_type=jnp.float32)
    out = out + jnp.dot(agg * inv, wn_ref[...], preferred_element_type=jnp.float32)
    out = out + b_ref[...]
    if act:
        out = jnp.maximum(out, 0.0)
    o_ref[...] = out


def _combine(h, agg2, deg_r, Ws, Wn, b, act):
    return pl.pallas_call(
        functools.partial(_combine_body, act),
        out_shape=jax.ShapeDtypeStruct((N, D), jnp.float32),
        grid=(GRID,),
        in_specs=[
            pl.BlockSpec((RB, D), lambda i: (i, 0)),
            pl.BlockSpec((NC, RB, D), lambda i: (0, i, 0)),
            pl.BlockSpec((1, NC, RB), lambda i: (i, 0, 0)),
            pl.BlockSpec((D, D), lambda i: (0, 0)),
            pl.BlockSpec((D, D), lambda i: (0, 0)),
            pl.BlockSpec((1, D), lambda i: (0, 0)),
        ],
        out_specs=pl.BlockSpec((RB, D), lambda i: (i, 0)),
    )(h, agg2, deg_r, Ws, Wn, b)


def kernel(features, edge_index, Wself0, Wneigh0, b0, Wself1, Wneigh1, b1,
           Wself2, Wneigh2, b2):
    src = edge_index[0]
    dst = edge_index[1]
    pad = E_PAD - E
    ar = jnp.arange(pad, dtype=src.dtype)
    src_p = jnp.concatenate([src, ar % N])
    dst_p = jnp.concatenate([dst, N + ar % (N_PAD - N)])

    agg0, deg2 = _get_sc_agg(True)(features, src_p, dst_p)
    deg_r = deg2[:, 0, :N].reshape(NC, GRID, RB).transpose(1, 0, 2)

    h1 = _combine(features, agg0, deg_r, Wself0, Wneigh0, b0.reshape(1, D), True)
    agg1 = _get_sc_agg(False)(h1, src_p, dst_p)
    h2 = _combine(h1, agg1, deg_r, Wself1, Wneigh1, b1.reshape(1, D), True)
    agg2 = _get_sc_agg(False)(h2, src_p, dst_p)
    h3 = _combine(h2, agg2, deg_r, Wself2, Wneigh2, b2.reshape(1, D), False)
    return h3


# speedup vs baseline: 4.1985x; 1.1731x over previous
import functools

import jax
import jax.numpy as jnp
from jax import lax
from jax.experimental import pallas as pl
from jax.experimental.pallas import tpu as pltpu
from jax.experimental.pallas import tpu_sc as plsc

N = 10000
E = 320000
D = 128
NC = 2
NS = 16
NW = NC * NS

CHUNK = 128
PCHUNKS = 2560
CPW = PCHUNKS // NW
PAIRS = CPW // 2
E_PAD = PCHUNKS * CHUNK

N_PAD = 10112
ROWS_TILE = N_PAD // NS
OUT_LAST = N - ROWS_TILE * (NS - 1)
N_DEG = 10240
DEG_SLICE = N_DEG // NS


def _make_sc_agg(with_deg):
    if with_deg:
        out_type = (jax.ShapeDtypeStruct((NC, N, D), jnp.float32),
                    jax.ShapeDtypeStruct((NC, 1, N_DEG), jnp.float32))
    else:
        out_type = jax.ShapeDtypeStruct((NC, N, D), jnp.float32)

    scratch = [
        pltpu.VMEM((CHUNK, D), jnp.float32),
        pltpu.VMEM((CHUNK, D), jnp.float32),
        pltpu.VMEM((CHUNK,), jnp.int32),
        pltpu.VMEM((CHUNK,), jnp.int32),
        pltpu.VMEM((CHUNK,), jnp.int32),
        pltpu.VMEM((CHUNK,), jnp.int32),
        pltpu.VMEM((CHUNK,), jnp.int32),
        pltpu.VMEM((CHUNK,), jnp.int32),
        pltpu.VMEM((CHUNK,), jnp.int32),
        pltpu.VMEM((CHUNK,), jnp.int32),
        pltpu.VMEM_SHARED((N_PAD, D), jnp.float32),
        pltpu.SemaphoreType.DMA,
        pltpu.SemaphoreType.DMA,
        pltpu.SemaphoreType.DMA,
        pltpu.SemaphoreType.DMA,
        pltpu.SemaphoreType.DMA,
        pltpu.SemaphoreType.DMA,
    ]
    if with_deg:
        scratch += [
            pltpu.VMEM((CHUNK,), jnp.float32),
            pltpu.VMEM((DEG_SLICE,), jnp.float32),
            pltpu.VMEM_SHARED((N_DEG,), jnp.float32),
        ]

    mesh = plsc.VectorSubcoreMesh(core_axis_name="c", subcore_axis_name="s")

    @functools.partial(pl.kernel, out_type=out_type, mesh=mesh,
                       scratch_types=scratch)
    def sc_agg(*refs):
        if with_deg:
            (h_hbm, src_hbm, dst_hbm, agg_hbm, deg_hbm,
             rb0, rb1, s0, d0, s1, d1, s2, d2, s3, d3, agg_sh,
             gsem0, gsem1, semi0, semi1, semi2, semi3,
             onesbuf, zdbuf, deg_sh) = refs
        else:
            (h_hbm, src_hbm, dst_hbm, agg_hbm,
             rb0, rb1, s0, d0, s1, d1, s2, d2, s3, d3, agg_sh,
             gsem0, gsem1, semi0, semi1, semi2, semi3) = refs
        isets = [(s0, d0, semi0), (s1, d1, semi1),
                 (s2, d2, semi2), (s3, d3, semi3)]

        c = lax.axis_index("c")
        s = lax.axis_index("s")
        wid = s * NC + c
        base = wid * CPW

        def idx_load(j, sbuf, dbuf, semx):
            off = (base + j) * CHUNK
            pltpu.async_copy(src_hbm.at[pl.ds(off, CHUNK)], sbuf, semx)
            pltpu.async_copy(dst_hbm.at[pl.ds(off, CHUNK)], dbuf, semx)

        def idx_wait(j, sbuf, dbuf, semx):
            off = (base + j) * CHUNK
            pltpu.make_async_copy(src_hbm.at[pl.ds(off, CHUNK)], sbuf, semx).wait()
            pltpu.make_async_copy(dst_hbm.at[pl.ds(off, CHUNK)], dbuf, semx).wait()

        for j in range(4):
            idx_load(j, *isets[j])

        zv = jnp.zeros((16,), jnp.float32)

        def zrow(r, carry):
            for j in range(D // 16):
                rb0[r, pl.ds(j * 16, 16)] = zv
            return carry

        lax.fori_loop(0, CHUNK, zrow, 0)

        if with_deg:
            ov = jnp.ones((16,), jnp.float32)
            for j in range(CHUNK // 16):
                onesbuf[pl.ds(j * 16, 16)] = ov
            for j in range(DEG_SLICE // 16):
                zdbuf[pl.ds(j * 16, 16)] = zv

        full, rem = ROWS_TILE // CHUNK, ROWS_TILE % CHUNK
        for k in range(full):
            pltpu.sync_copy(rb0, agg_sh.at[pl.ds(s * ROWS_TILE + k * CHUNK, CHUNK), :])
        if rem:
            pltpu.sync_copy(rb0.at[pl.ds(0, rem), :],
                            agg_sh.at[pl.ds(s * ROWS_TILE + full * CHUNK, rem), :])
        if with_deg:
            pltpu.sync_copy(zdbuf, deg_sh.at[pl.ds(s * DEG_SLICE, DEG_SLICE)])

        rbs = [rb0, rb1]
        gsems = [gsem0, gsem1]
        idx_wait(0, *isets[0])
        pltpu.async_copy(h_hbm.at[s0], rb0, gsem0)
        idx_wait(1, *isets[1])
        pltpu.async_copy(h_hbm.at[s1], rb1, gsem1)

        plsc.subcore_barrier()

        def quad(q, carry):
            c0 = 4 * q
            for k in range(4):
                ck = c0 + k
                sk, dk, semk = isets[k]
                rbk = rbs[k % 2]
                gsk = gsems[k % 2]
                pltpu.make_async_copy(h_hbm.at[sk], rbk, gsk).wait()
                pltpu.sync_copy(rbk, agg_sh.at[dk], add=True)
                if with_deg:
                    pltpu.sync_copy(onesbuf, deg_sh.at[dk], add=True)

                @pl.when(ck + 2 < CPW)
                def _():
                    sn, dn, semn = isets[(k + 2) % 4]
                    idx_wait(ck + 2, sn, dn, semn)
                    pltpu.async_copy(h_hbm.at[sn], rbk, gsk)

                @pl.when(ck + 4 < CPW)
                def _():
                    idx_load(ck + 4, sk, dk, semk)
            return carry

        lax.fori_loop(0, CPW // 4, quad, 0)

        plsc.subcore_barrier()

        @pl.when(s < NS - 1)
        def _():
            pltpu.sync_copy(agg_sh.at[pl.ds(s * ROWS_TILE, ROWS_TILE), :],
                            agg_hbm.at[c, pl.ds(s * ROWS_TILE, ROWS_TILE), :])

        @pl.when(s == NS - 1)
        def _():
            pltpu.sync_copy(agg_sh.at[pl.ds((NS - 1) * ROWS_TILE, OUT_LAST), :],
                            agg_hbm.at[c, pl.ds((NS - 1) * ROWS_TILE, OUT_LAST), :])

        if with_deg:
            pltpu.sync_copy(deg_sh.at[pl.ds(s * DEG_SLICE, DEG_SLICE)],
                            deg_hbm.at[c, 0, pl.ds(s * DEG_SLICE, DEG_SLICE)])

    return sc_agg


@functools.lru_cache(maxsize=None)
def _get_sc_agg(with_deg):
    return _make_sc_agg(with_deg)


RB = 1000
GRID = N // RB


def _combine_body(act, h_ref, agg_ref, deg_ref, ws_ref, wn_ref, b_ref, o_ref):
    deg = jnp.sum(deg_ref[...], axis=(0, 1))
    inv = (1.0 / jnp.maximum(deg, 1.0))[:, None]
    agg = agg_ref[0] + agg_ref[1]
    out = jnp.dot(h_ref[...], ws_ref[...], preferred_element_type=jnp.float32)
    out = out + jnp.dot(agg * inv, wn_ref[...], preferred_element_type=jnp.float32)
    out = out + b_ref[...]
    if act:
        out = jnp.maximum(out, 0.0)
    o_ref[...] = out


def _combine(h, agg2, deg_r, Ws, Wn, b, act):
    return pl.pallas_call(
        functools.partial(_combine_body, act),
        out_shape=jax.ShapeDtypeStruct((N, D), jnp.float32),
        grid=(GRID,),
        in_specs=[
            pl.BlockSpec((RB, D), lambda i: (i, 0)),
            pl.BlockSpec((NC, RB, D), lambda i: (0, i, 0)),
            pl.BlockSpec((1, NC, RB), lambda i: (i, 0, 0)),
            pl.BlockSpec((D, D), lambda i: (0, 0)),
            pl.BlockSpec((D, D), lambda i: (0, 0)),
            pl.BlockSpec((1, D), lambda i: (0, 0)),
        ],
        out_specs=pl.BlockSpec((RB, D), lambda i: (i, 0)),
    )(h, agg2, deg_r, Ws, Wn, b)


def kernel(features, edge_index, Wself0, Wneigh0, b0, Wself1, Wneigh1, b1,
           Wself2, Wneigh2, b2):
    src = edge_index[0]
    dst = edge_index[1]
    pad = E_PAD - E
    ar = jnp.arange(pad, dtype=src.dtype)
    src_p = jnp.concatenate([src, ar % N])
    dst_p = jnp.concatenate([dst, N + ar % (N_PAD - N)])

    agg0, deg2 = _get_sc_agg(True)(features, src_p, dst_p)
    deg_r = deg2[:, 0, :N].reshape(NC, GRID, RB).transpose(1, 0, 2)

    h1 = _combine(features, agg0, deg_r, Wself0, Wneigh0, b0.reshape(1, D), True)
    agg1 = _get_sc_agg(False)(h1, src_p, dst_p)
    h2 = _combine(h1, agg1, deg_r, Wself1, Wneigh1, b1.reshape(1, D), True)
    agg2 = _get_sc_agg(False)(h2, src_p, dst_p)
    h3 = _combine(h2, agg2, deg_r, Wself2, Wneigh2, b2.reshape(1, D), False)
    return h3
